# baseline (speedup 1.0000x reference)
import functools

import jax
import jax.numpy as jnp
from jax import lax
from jax.experimental import pallas as pl
from jax.experimental.pallas import tpu as pltpu
from jax.experimental.pallas import tpu_sc as plsc

N = 10000
E = 160000
NH = 256
NS = 16
NC = 2
SP = 8
B = 128
SPAN = 4992
NBATCH = SPAN // B
TAIL0 = 32 * SPAN
ZROWS = N * SP // NS
ZCH = 200
CW = 256

_mesh = lambda: plsc.VectorSubcoreMesh(core_axis_name="c", subcore_axis_name="s")


def _zero_half(c, s, zbuf, out_hbm):
    base = c * N * SP + s * ZROWS
    for k in range(ZROWS // ZCH):
        pltpu.sync_copy(zbuf, out_hbm.at[pl.ds(base + k * ZCH, ZCH)])


def _shift_spread(idx_ref, base, w):
    lane = lax.broadcasted_iota(jnp.int32, (16,), 0)
    for j in range(B // 16):
        v = idx_ref[pl.ds(j * 16, 16)]
        idx_ref[pl.ds(j * 16, 16)] = (v + base) * SP + ((lane + w) & 7)


def _sc_segsum_kernel(gathered):
    def body(table, src, dst, z, *rest):
        if gathered:
            (out0, dstb, srcb, rows, zbuf, gsem, ssem) = rest
            outs = (out0,)
            ones_v = None
        else:
            (ones16, es_out, in_out, out_out, dstb, srcb, rows, ones_v,
             zbuf, gsem, ssem) = rest
            outs = (es_out, in_out, out_out)

        c = lax.axis_index("c")
        s = lax.axis_index("s")
        t = c * NS + s
        base = c * N
        pltpu.sync_copy(z, zbuf)
        for o in outs:
            _zero_half(c, s, zbuf, o)
        if not gathered:
            pltpu.sync_copy(ones16, ones_v)
        plsc.subcore_barrier()

        def load_batch(off, slot):
            pltpu.sync_copy(dst.at[pl.ds(off, B)], dstb)
            pltpu.sync_copy(src.at[pl.ds(off, B)], srcb)
            if gathered:
                pltpu.async_copy(table.at[srcb], rows, gsem).wait()
            else:
                pltpu.sync_copy(table.at[pl.ds(off, B)], rows)
            _shift_spread(dstb, base, slot)
            if not gathered:
                _shift_spread(srcb, base, slot)

        def scatter_batch():
            pltpu.async_copy(rows, outs[0].at[dstb], ssem, add=True).wait()
            if not gathered:
                pltpu.async_copy(ones_v, outs[1].at[dstb], ssem,
                                 add=True).wait()
                pltpu.async_copy(ones_v, outs[2].at[srcb], ssem,
                                 add=True).wait()

        def wave(w, carry):
            load_batch(t * SPAN + w * B, w & 7)
            for k in range(NS):
                @pl.when(s == k)
                def _():
                    scatter_batch()
                plsc.subcore_barrier()
            return carry

        lax.fori_loop(0, NBATCH, wave, 0)

        @pl.when(t < 2)
        def _():
            load_batch(TAIL0 + t * B, (NBATCH + t) & 7)
        for k in range(2):
            @pl.when(t == k)
            def _():
                scatter_batch()
            plsc.subcore_barrier()

    return body


_sc_prelude = functools.partial(
    pl.kernel,
    out_type=[
        jax.ShapeDtypeStruct((NC * N * SP, NH), jnp.float32),
        jax.ShapeDtypeStruct((NC * N * SP, CW), jnp.float32),
        jax.ShapeDtypeStruct((NC * N * SP, CW), jnp.float32),
    ],
    mesh=_mesh(),
    scratch_types=[
        pltpu.VMEM((B,), jnp.int32),
        pltpu.VMEM((B,), jnp.int32),
        pltpu.VMEM((B, NH), jnp.float32),
        pltpu.VMEM((B, CW), jnp.float32),
        pltpu.VMEM((ZCH, NH), jnp.float32),
        pltpu.SemaphoreType.DMA,
        pltpu.SemaphoreType.DMA,
    ],
)(_sc_segsum_kernel(gathered=False))


_sc_nbsum = functools.partial(
    pl.kernel,
    out_type=jax.ShapeDtypeStruct((NC * N * SP, NH), jnp.float32),
    mesh=_mesh(),
    scratch_types=[
        pltpu.VMEM((B,), jnp.int32),
        pltpu.VMEM((B,), jnp.int32),
        pltpu.VMEM((B, NH), jnp.float32),
        pltpu.VMEM((ZCH, NH), jnp.float32),
        pltpu.SemaphoreType.DMA,
        pltpu.SemaphoreType.DMA,
    ],
)(_sc_segsum_kernel(gathered=True))

_sc_rsum = _sc_nbsum


R = 1000
HOFF8 = N * SP // (SP * R)


def _reduce_body(a_ref, b_ref, o_ref):
    a = a_ref[...].reshape(R, SP, NH)
    b = b_ref[...].reshape(R, SP, NH)
    o_ref[...] = jnp.sum(a, axis=1) + jnp.sum(b, axis=1)


_reduce_tc = pl.pallas_call(
    _reduce_body,
    grid=(N // R,),
    in_specs=[
        pl.BlockSpec((SP * R, NH), lambda i: (i, 0)),
        pl.BlockSpec((SP * R, NH), lambda i: (i + HOFF8, 0)),
    ],
    out_specs=pl.BlockSpec((R, NH), lambda i: (i, 0)),
    out_shape=jax.ShapeDtypeStruct((N, NH), jnp.float32),
)


def _round_body(h_ref, nb_ref, es_ref, in_ref,
                wm_ref, bm_ref, wih_ref, whh_ref, bih_ref, bhh_ref, o_ref):
    h = h_ref[...]
    cnt = in_ref[...][:, 0:1]
    nb = nb_ref[...]
    es = es_ref[...]
    wm = wm_ref[...]
    a_self = jnp.dot(h, wm[0:NH], preferred_element_type=jnp.float32)
    a_nb = jnp.dot(nb, wm[NH:2 * NH], preferred_element_type=jnp.float32)
    a_es = jnp.dot(es, wm[2 * NH:3 * NH], preferred_element_type=jnp.float32)
    act = (jnp.where(cnt > 0.0, a_self + bm_ref[...], 0.0)
           + (a_nb + a_es) / jnp.maximum(cnt, 1.0))
    gi = jnp.dot(act, wih_ref[...], preferred_element_type=jnp.float32) + bih_ref[...]
    gh = jnp.dot(h, whh_ref[...], preferred_element_type=jnp.float32) + bhh_ref[...]
    r = jax.nn.sigmoid(gi[:, 0:NH] + gh[:, 0:NH])
    z = jax.nn.sigmoid(gi[:, NH:2 * NH] + gh[:, NH:2 * NH])
    n = jnp.tanh(gi[:, 2 * NH:] + r * gh[:, 2 * NH:])
    o_ref[...] = (1.0 - z) * n + z * h


_round_tc = pl.pallas_call(
    _round_body,
    grid=(N // R,),
    in_specs=[
        pl.BlockSpec((R, NH), lambda i: (i, 0)),
        pl.BlockSpec((R, NH), lambda i: (i, 0)),
        pl.BlockSpec((R, NH), lambda i: (i, 0)),
        pl.BlockSpec((R, CW), lambda i: (i, 0)),
        pl.BlockSpec((3 * NH, 2 * NH), lambda i: (0, 0)),
        pl.BlockSpec((1, 2 * NH), lambda i: (0, 0)),
        pl.BlockSpec((2 * NH, 3 * NH), lambda i: (0, 0)),
        pl.BlockSpec((NH, 3 * NH), lambda i: (0, 0)),
        pl.BlockSpec((1, 3 * NH), lambda i: (0, 0)),
        pl.BlockSpec((1, 3 * NH), lambda i: (0, 0)),
    ],
    out_specs=pl.BlockSpec((R, NH), lambda i: (i, 0)),
    out_shape=jax.ShapeDtypeStruct((N, NH), jnp.float32),
)


def _feat_body(h_ref, od_ref, wg_ref, bg_ref, o_ref):
    g = jax.nn.sigmoid(
        jnp.dot(h_ref[...], wg_ref[...], preferred_element_type=jnp.float32)
        + bg_ref[...])
    od = od_ref[...][:, 0:1]
    ns = lax.rsqrt(jnp.maximum(od, 1.0))
    colmask = lax.broadcasted_iota(jnp.int32, (R, CW), 1) < 7
    o_ref[...] = jnp.where(colmask, g * ns, 0.0)


_feat_tc = pl.pallas_call(
    _feat_body,
    grid=(N // R,),
    in_specs=[
        pl.BlockSpec((R, NH), lambda i: (i, 0)),
        pl.BlockSpec((R, CW), lambda i: (i, 0)),
        pl.BlockSpec((NH, CW), lambda i: (0, 0)),
        pl.BlockSpec((1, CW), lambda i: (0, 0)),
    ],
    out_specs=pl.BlockSpec((R, CW), lambda i: (i, 0)),
    out_shape=jax.ShapeDtypeStruct((N, CW), jnp.float32),
)


def _final_body(r_ref, in_ref, wc_ref, bc_ref, o_ref):
    cnt = in_ref[...][:, 0:1]
    nd = lax.rsqrt(jnp.maximum(cnt, 1.0))
    sv = jnp.sum(r_ref[...] * nd, axis=0, keepdims=True)
    o_ref[...] = (jnp.dot(sv, wc_ref[...], preferred_element_type=jnp.float32)
                  + float(N) * bc_ref[...])


_final_tc = pl.pallas_call(
    _final_body,
    grid=(1,),
    in_specs=[
        pl.BlockSpec((N, CW), lambda i: (0, 0)),
        pl.BlockSpec((N, CW), lambda i: (0, 0)),
        pl.BlockSpec((CW, 2 * NH), lambda i: (0, 0)),
        pl.BlockSpec((1, 2 * NH), lambda i: (0, 0)),
    ],
    out_specs=pl.BlockSpec((1, 2 * NH), lambda i: (0, 0)),
    out_shape=jax.ShapeDtypeStruct((1, 2 * NH), jnp.float32),
)


def kernel(x, edge_index, edge_attr, W_msg_0, b_msg_0, W_ih_0, W_hh_0,
           b_ih_0, b_hh_0, W_msg_1, b_msg_1, W_ih_1, W_hh_1, b_ih_1,
           b_hh_1, W_gate, b_gate, W_conv, b_conv):
    src = edge_index[0]
    dst = edge_index[1]
    z256 = jnp.zeros((ZCH, NH), jnp.float32)
    ones16 = jnp.zeros((B, CW), jnp.float32).at[:, 0].set(1.0)

    es8, in8, out8 = _sc_prelude(edge_attr, src, dst, z256, ones16)
    es = _reduce_tc(es8, es8)
    cnt = _reduce_tc(in8, in8)
    od = _reduce_tc(out8, out8)

    h = x
    params = [(W_msg_0, b_msg_0, W_ih_0, W_hh_0, b_ih_0, b_hh_0),
              (W_msg_1, b_msg_1, W_ih_1, W_hh_1, b_ih_1, b_hh_1)]
    for (wm, bm, wih, whh, bih, bhh) in params:
        nb8 = _sc_nbsum(h, src, dst, z256)
        nb = _reduce_tc(nb8, nb8)
        h = _round_tc(h, nb, es, cnt,
                      wm, bm.reshape(1, -1), wih, whh,
                      bih.reshape(1, -1), bhh.reshape(1, -1))

    wg16 = jnp.pad(W_gate, ((0, 0), (0, CW - 7)))
    bg16 = jnp.pad(b_gate, (0, CW - 7)).reshape(1, CW)
    feat = _feat_tc(h, od, wg16, bg16)
    r8 = _sc_rsum(feat, src, dst, z256)
    racc = _reduce_tc(r8, r8)
    wc16 = jnp.pad(W_conv, ((0, CW - 7), (0, 0)))
    return _final_tc(racc, cnt, wc16, b_conv.reshape(1, -1))


# speedup vs baseline: 1.2779x; 1.2779x over previous
import functools

import jax
import jax.numpy as jnp
from jax import lax
from jax.experimental import pallas as pl
from jax.experimental.pallas import tpu as pltpu
from jax.experimental.pallas import tpu_sc as plsc

N = 10000
E = 160000
NH = 256
PW = NH + 128
NS = 16
NC = 2
SP = 16
B = 128
SPAN = 4992
NBATCH = SPAN // B
TAIL0 = 32 * SPAN
ZROWS = N * SP // NS
ZCH = 104
CW = 256

_mesh = lambda: plsc.VectorSubcoreMesh(core_axis_name="c", subcore_axis_name="s")


def _zero_half(c, s, zbuf, out_hbm):
    base = c * N * SP + s * ZROWS
    nfull = ZROWS // ZCH
    for k in range(nfull):
        pltpu.sync_copy(zbuf, out_hbm.at[pl.ds(base + k * ZCH, ZCH)])
    rem = ZROWS - nfull * ZCH
    if rem:
        pltpu.sync_copy(zbuf.at[pl.ds(0, rem)],
                        out_hbm.at[pl.ds(base + nfull * ZCH, rem)])


def _shift_spread(idx_ref, base, w, grp):
    lane = lax.broadcasted_iota(jnp.int32, (16,), 0)
    for j in range(B // 16):
        v = idx_ref[pl.ds(j * 16, 16)]
        idx_ref[pl.ds(j * 16, 16)] = (v + base) * SP + 8 * grp + ((lane + w) & 7)


def _sc_segsum_kernel(gathered):
    def body(table, src, dst, z, *rest):
        if gathered:
            (out0, dstb, srcb, rows, zbuf, gsem, ssem) = rest
        else:
            (onescol, ones16, pk_out, out_out, dstb, srcb, rows, ones_v,
             zbuf, gsem, ssem) = rest

        c = lax.axis_index("c")
        s = lax.axis_index("s")
        t = c * NS + s
        base = c * N
        pltpu.sync_copy(z, zbuf)
        if gathered:
            _zero_half(c, s, zbuf, out0)
        else:
            _zero_half(c, s, zbuf.at[:, pl.ds(0, PW)], pk_out)
            _zero_half(c, s, zbuf.at[:, pl.ds(0, CW)], out_out)
            pltpu.sync_copy(ones16, ones_v)
            pltpu.sync_copy(onescol, rows.at[:, pl.ds(NH, 128)])
        plsc.subcore_barrier()

        def issue_load(off):
            pltpu.sync_copy(dst.at[pl.ds(off, B)], dstb)
            pltpu.sync_copy(src.at[pl.ds(off, B)], srcb)
            if gathered:
                pltpu.async_copy(table.at[srcb], rows, gsem)
            else:
                pltpu.async_copy(table.at[pl.ds(off, B)],
                                 rows.at[:, pl.ds(0, NH)], gsem)

        def drain_load(off):
            if gathered:
                pltpu.make_async_copy(table.at[srcb], rows, gsem).wait()
            else:
                pltpu.make_async_copy(table.at[pl.ds(off, B)],
                                      rows.at[:, pl.ds(0, NH)], gsem).wait()

        def scatter_batch(w):
            grp = s & 1
            _shift_spread(dstb, base, w, grp)
            if gathered:
                pltpu.async_copy(rows, out0.at[dstb], ssem, add=True).wait()
            else:
                _shift_spread(srcb, base, w, grp)
                pltpu.async_copy(rows, pk_out.at[dstb], ssem, add=True).wait()
                pltpu.async_copy(ones_v, out_out.at[srcb], ssem,
                                 add=True).wait()

        issue_load(t * SPAN)

        def wave(w, carry):
            for k in range(NS // 2):
                @pl.when((s >> 1) == k)
                def _():
                    drain_load(t * SPAN + w * B)
                    scatter_batch(w)

                    @pl.when(w + 1 < NBATCH)
                    def _():
                        issue_load(t * SPAN + (w + 1) * B)
                plsc.subcore_barrier()
            return carry

        lax.fori_loop(0, NBATCH, wave, 0)

        @pl.when(t < 2)
        def _():
            issue_load(TAIL0 + t * B)
        @pl.when(t < 2)
        def _():
            drain_load(TAIL0 + t * B)
            scatter_batch(NBATCH + t)
        plsc.subcore_barrier()

    return body


_sc_prelude = functools.partial(
    pl.kernel,
    out_type=[
        jax.ShapeDtypeStruct((NC * N * SP, PW), jnp.float32),
        jax.ShapeDtypeStruct((NC * N * SP, CW), jnp.float32),
    ],
    mesh=_mesh(),
    scratch_types=[
        pltpu.VMEM((B,), jnp.int32),
        pltpu.VMEM((B,), jnp.int32),
        pltpu.VMEM((B, PW), jnp.float32),
        pltpu.VMEM((B, CW), jnp.float32),
        pltpu.VMEM((ZCH, PW), jnp.float32),
        pltpu.SemaphoreType.DMA,
        pltpu.SemaphoreType.DMA,
    ],
)(_sc_segsum_kernel(gathered=False))


_sc_nbsum = functools.partial(
    pl.kernel,
    out_type=jax.ShapeDtypeStruct((NC * N * SP, NH), jnp.float32),
    mesh=_mesh(),
    scratch_types=[
        pltpu.VMEM((B,), jnp.int32),
        pltpu.VMEM((B,), jnp.int32),
        pltpu.VMEM((B, NH), jnp.float32),
        pltpu.VMEM((ZCH, NH), jnp.float32),
        pltpu.SemaphoreType.DMA,
        pltpu.SemaphoreType.DMA,
    ],
)(_sc_segsum_kernel(gathered=True))

_sc_rsum = _sc_nbsum


R = 1000
RR = 200
HOFF8 = N // RR


def _mk_reduce(width):
    def _reduce_body(a_ref, b_ref, o_ref):
        a = a_ref[...].reshape(RR, SP, width)
        b = b_ref[...].reshape(RR, SP, width)
        o_ref[...] = jnp.sum(a, axis=1) + jnp.sum(b, axis=1)

    return pl.pallas_call(
        _reduce_body,
        grid=(N // RR,),
        in_specs=[
            pl.BlockSpec((SP * RR, width), lambda i: (i, 0)),
            pl.BlockSpec((SP * RR, width), lambda i: (i + HOFF8, 0)),
        ],
        out_specs=pl.BlockSpec((RR, width), lambda i: (i, 0)),
        out_shape=jax.ShapeDtypeStruct((N, width), jnp.float32),
    )


_reduce_nh = _mk_reduce(NH)
_reduce_pw = _mk_reduce(PW)
_reduce_cw = _mk_reduce(CW)


def _round_body(h_ref, nb_ref, pk_ref,
                wm_ref, bm_ref, wih_ref, whh_ref, bih_ref, bhh_ref, o_ref):
    h = h_ref[...]
    pk = pk_ref[...]
    cnt = pk[:, NH:NH + 1]
    es = pk[:, 0:NH]
    nb = nb_ref[...]
    wm = wm_ref[...]
    a_self = jnp.dot(h, wm[0:NH], preferred_element_type=jnp.float32)
    a_nb = jnp.dot(nb, wm[NH:2 * NH], preferred_element_type=jnp.float32)
    a_es = jnp.dot(es, wm[2 * NH:3 * NH], preferred_element_type=jnp.float32)
    act = (jnp.where(cnt > 0.0, a_self + bm_ref[...], 0.0)
           + (a_nb + a_es) / jnp.maximum(cnt, 1.0))
    gi = jnp.dot(act, wih_ref[...], preferred_element_type=jnp.float32) + bih_ref[...]
    gh = jnp.dot(h, whh_ref[...], preferred_element_type=jnp.float32) + bhh_ref[...]
    r = jax.nn.sigmoid(gi[:, 0:NH] + gh[:, 0:NH])
    z = jax.nn.sigmoid(gi[:, NH:2 * NH] + gh[:, NH:2 * NH])
    n = jnp.tanh(gi[:, 2 * NH:] + r * gh[:, 2 * NH:])
    o_ref[...] = (1.0 - z) * n + z * h


_round_tc = pl.pallas_call(
    _round_body,
    grid=(N // R,),
    in_specs=[
        pl.BlockSpec((R, NH), lambda i: (i, 0)),
        pl.BlockSpec((R, NH), lambda i: (i, 0)),
        pl.BlockSpec((R, PW), lambda i: (i, 0)),
        pl.BlockSpec((3 * NH, 2 * NH), lambda i: (0, 0)),
        pl.BlockSpec((1, 2 * NH), lambda i: (0, 0)),
        pl.BlockSpec((2 * NH, 3 * NH), lambda i: (0, 0)),
        pl.BlockSpec((NH, 3 * NH), lambda i: (0, 0)),
        pl.BlockSpec((1, 3 * NH), lambda i: (0, 0)),
        pl.BlockSpec((1, 3 * NH), lambda i: (0, 0)),
    ],
    out_specs=pl.BlockSpec((R, NH), lambda i: (i, 0)),
    out_shape=jax.ShapeDtypeStruct((N, NH), jnp.float32),
)


def _feat_body(h_ref, od_ref, wg_ref, bg_ref, o_ref):
    g = jax.nn.sigmoid(
        jnp.dot(h_ref[...], wg_ref[...], preferred_element_type=jnp.float32)
        + bg_ref[...])
    od = od_ref[...][:, 0:1]
    ns = lax.rsqrt(jnp.maximum(od, 1.0))
    colmask = lax.broadcasted_iota(jnp.int32, (R, CW), 1) < 7
    o_ref[...] = jnp.where(colmask, g * ns, 0.0)


_feat_tc = pl.pallas_call(
    _feat_body,
    grid=(N // R,),
    in_specs=[
        pl.BlockSpec((R, NH), lambda i: (i, 0)),
        pl.BlockSpec((R, CW), lambda i: (i, 0)),
        pl.BlockSpec((NH, CW), lambda i: (0, 0)),
        pl.BlockSpec((1, CW), lambda i: (0, 0)),
    ],
    out_specs=pl.BlockSpec((R, CW), lambda i: (i, 0)),
    out_shape=jax.ShapeDtypeStruct((N, CW), jnp.float32),
)


def _final_body(r_ref, pk_ref, wc_ref, bc_ref, o_ref):
    cnt = pk_ref[...][:, NH:NH + 1]
    nd = lax.rsqrt(jnp.maximum(cnt, 1.0))
    sv = jnp.sum(r_ref[...] * nd, axis=0, keepdims=True)
    o_ref[...] = (jnp.dot(sv, wc_ref[...], preferred_element_type=jnp.float32)
                  + float(N) * bc_ref[...])


_final_tc = pl.pallas_call(
    _final_body,
    grid=(1,),
    in_specs=[
        pl.BlockSpec((N, CW), lambda i: (0, 0)),
        pl.BlockSpec((N, PW), lambda i: (0, 0)),
        pl.BlockSpec((CW, 2 * NH), lambda i: (0, 0)),
        pl.BlockSpec((1, 2 * NH), lambda i: (0, 0)),
    ],
    out_specs=pl.BlockSpec((1, 2 * NH), lambda i: (0, 0)),
    out_shape=jax.ShapeDtypeStruct((1, 2 * NH), jnp.float32),
)


def kernel(x, edge_index, edge_attr, W_msg_0, b_msg_0, W_ih_0, W_hh_0,
           b_ih_0, b_hh_0, W_msg_1, b_msg_1, W_ih_1, W_hh_1, b_ih_1,
           b_hh_1, W_gate, b_gate, W_conv, b_conv):
    src = edge_index[0]
    dst = edge_index[1]
    zpk = jnp.zeros((ZCH, PW), jnp.float32)
    znh = jnp.zeros((ZCH, NH), jnp.float32)
    onescol = jnp.zeros((B, 128), jnp.float32).at[:, 0].set(1.0)
    ones16 = jnp.zeros((B, CW), jnp.float32).at[:, 0].set(1.0)

    pk8, out8 = _sc_prelude(edge_attr, src, dst, zpk, onescol, ones16)
    pk = _reduce_pw(pk8, pk8)
    od = _reduce_cw(out8, out8)

    h = x
    params = [(W_msg_0, b_msg_0, W_ih_0, W_hh_0, b_ih_0, b_hh_0),
              (W_msg_1, b_msg_1, W_ih_1, W_hh_1, b_ih_1, b_hh_1)]
    for (wm, bm, wih, whh, bih, bhh) in params:
        nb8 = _sc_nbsum(h, src, dst, znh)
        nb = _reduce_nh(nb8, nb8)
        h = _round_tc(h, nb, pk,
                      wm, bm.reshape(1, -1), wih, whh,
                      bih.reshape(1, -1), bhh.reshape(1, -1))

    wg16 = jnp.pad(W_gate, ((0, 0), (0, CW - 7)))
    bg16 = jnp.pad(b_gate, (0, CW - 7)).reshape(1, CW)
    feat = _feat_tc(h, od, wg16, bg16)
    r8 = _sc_rsum(feat, src, dst, znh)
    racc = _reduce_cw(r8, r8)
    wc16 = jnp.pad(W_conv, ((0, CW - 7), (0, 0)))
    return _final_tc(racc, pk, wc16, b_conv.reshape(1, -1))
